# baseline (speedup 1.0000x reference)
import math

import jax
import jax.numpy as jnp
from jax.experimental import pallas as pl
from jax.experimental.pallas import tpu as pltpu

_G = 4
_MAXLEN = 128
_B = 64
_NB = 1000
_EBE = 256
_EBO = 512
_RB = 512


def _mm_kernel(x_ref, w_ref, o_ref):
    o_ref[...] = jnp.dot(x_ref[...], w_ref[...],
                         preferred_element_type=jnp.float32)


def _matmul(x, w):
    return pl.pallas_call(
        _mm_kernel,
        out_shape=jax.ShapeDtypeStruct((x.shape[0], w.shape[1]), jnp.float32),
    )(x, w)


def _group_mat(d, dtype=jnp.float32):
    dg = d // _G
    gi = jax.lax.broadcasted_iota(jnp.int32, (d, _G), 0) // dg
    gj = jax.lax.broadcasted_iota(jnp.int32, (d, _G), 1)
    return (gi == gj).astype(dtype)


def _split_dot(oh_bf, table):
    hi = table.astype(jnp.bfloat16)
    lo = (table - hi.astype(jnp.float32)).astype(jnp.bfloat16)
    return (jnp.dot(oh_bf, hi, preferred_element_type=jnp.float32)
            + jnp.dot(oh_bf, lo, preferred_element_type=jnp.float32))


def _edge_kernel(ea_ref, src_ref, dst_ref, hwm_ref, xwq_ref, we_ref,
                 m_ref, lg_ref):
    n = hwm_ref.shape[0]
    eb = ea_ref.shape[0]
    d = hwm_ref.shape[1]
    srcv = src_ref[0]
    dstv = dst_ref[0]
    iota_n = jax.lax.broadcasted_iota(jnp.int32, (eb, n), 1)
    oh_s = (iota_n == srcv).astype(jnp.bfloat16)
    oh_d = (iota_n == dstv).astype(jnp.bfloat16)
    e = jnp.dot(ea_ref[...], we_ref[...], preferred_element_type=jnp.float32)
    m = _split_dot(oh_s, hwm_ref[...]) + e
    q = _split_dot(oh_d, xwq_ref[...])
    m_ref[...] = m
    rt = _group_mat(d)
    lg_ref[...] = jnp.dot(m * q, rt, preferred_element_type=jnp.float32) * (
        1.0 / math.sqrt(d // _G))


def _out_kernel(lg_ref, m_ref, dst_ref, h_ref, o_ref, sacc_ref, aacc_ref):
    e_idx = pl.program_id(1)
    n_e = pl.num_programs(1)
    nb = o_ref.shape[0]
    eb = m_ref.shape[0]
    d = m_ref.shape[1]

    @pl.when(e_idx == 0)
    def _():
        sacc_ref[...] = jnp.zeros_like(sacc_ref)
        aacc_ref[...] = jnp.zeros_like(aacc_ref)

    base = pl.program_id(0) * nb
    dstv = dst_ref[0]
    iota_nb = jax.lax.broadcasted_iota(jnp.int32, (nb, eb), 0)
    oh_bf = (iota_nb == (dstv - base)).astype(jnp.bfloat16)
    ex = jnp.exp(lg_ref[...])
    r = _group_mat(d)
    w = m_ref[...] * jnp.dot(ex, r.T, preferred_element_type=jnp.float32)
    aacc_ref[...] += _split_dot(oh_bf, w)
    sacc_ref[...] += _split_dot(oh_bf, ex)

    @pl.when(e_idx == n_e - 1)
    def _():
        sfull = jnp.dot(sacc_ref[...], r.T,
                        preferred_element_type=jnp.float32)
        out = aacc_ref[...] / (sfull + 1e-30)
        o_ref[...] = jnp.maximum(out + h_ref[...], 0.0)


def _scatter_kernel(lb_ref, v_ref, o_ref):
    lp = v_ref.shape[0]
    rb = o_ref.shape[0]
    base = pl.program_id(0) * rb
    lbv = lb_ref[0]
    bio = jax.lax.broadcasted_iota(jnp.int32, (_B, lp), 0)
    p = bio == lbv
    colidx = jax.lax.broadcasted_iota(jnp.int32, (_B, lp), 1)
    fo = jnp.min(jnp.where(p, colidx, lp), axis=1, keepdims=True)
    forow = jnp.sum(jnp.where(p, fo, 0), axis=0, keepdims=True)
    idxrow = jax.lax.broadcasted_iota(jnp.int32, (1, lp), 1)
    pos = idxrow - forow
    valid = (lbv < _B) & (pos < _MAXLEN)
    r = jnp.where(valid, lbv * _MAXLEN + pos, -1)
    rio = jax.lax.broadcasted_iota(jnp.int32, (rb, lp), 0) + base
    oh = (rio == r).astype(jnp.float32)
    o_ref[...] = jnp.dot(oh, v_ref[...], preferred_element_type=jnp.float32)


def kernel(x, edge_index, edge_attr, ligand_batch,
           Wm0, We0, Wq0, Wm1, We1, Wq1, Wm2, We2, Wq2):
    n, d = x.shape
    src, dst = edge_index[0], edge_index[1]
    src2 = jnp.concatenate([src, dst])
    dst2 = jnp.concatenate([dst, src])
    ea2 = jnp.concatenate([edge_attr, edge_attr], axis=0)
    e2 = src2.shape[0]
    de = ea2.shape[1]

    nbe = e2 // _EBE
    nbo = e2 // _EBO
    nbn = n // _NB
    src_c = src2.reshape(nbe, _EBE, 1)
    dst_c = dst2.reshape(nbe, _EBE, 1)
    dst_r = dst2.reshape(nbo, 1, _EBO)

    edge_call = pl.pallas_call(
        _edge_kernel,
        grid=(nbe,),
        in_specs=[
            pl.BlockSpec((_EBE, de), lambda i: (i, 0)),
            pl.BlockSpec((1, _EBE, 1), lambda i: (i, 0, 0)),
            pl.BlockSpec((1, _EBE, 1), lambda i: (i, 0, 0)),
            pl.BlockSpec((n, d), lambda i: (0, 0)),
            pl.BlockSpec((n, d), lambda i: (0, 0)),
            pl.BlockSpec((de, d), lambda i: (0, 0)),
        ],
        out_specs=[
            pl.BlockSpec((_EBE, d), lambda i: (i, 0)),
            pl.BlockSpec((_EBE, _G), lambda i: (i, 0)),
        ],
        out_shape=[
            jax.ShapeDtypeStruct((e2, d), jnp.float32),
            jax.ShapeDtypeStruct((e2, _G), jnp.float32),
        ],
    )

    out_call = pl.pallas_call(
        _out_kernel,
        grid=(nbn, nbo),
        in_specs=[
            pl.BlockSpec((_EBO, _G), lambda i, j: (j, 0)),
            pl.BlockSpec((_EBO, d), lambda i, j: (j, 0)),
            pl.BlockSpec((1, 1, _EBO), lambda i, j: (j, 0, 0)),
            pl.BlockSpec((_NB, d), lambda i, j: (i, 0)),
        ],
        out_specs=pl.BlockSpec((_NB, d), lambda i, j: (i, 0)),
        out_shape=jax.ShapeDtypeStruct((n, d), jnp.float32),
        scratch_shapes=[
            pltpu.VMEM((_NB, _G), jnp.float32),
            pltpu.VMEM((_NB, d), jnp.float32),
        ],
    )

    h = x
    for wm, we, wq in ((Wm0, We0, Wq0), (Wm1, We1, Wq1), (Wm2, We2, Wq2)):
        hwm = _matmul(h, wm)
        xwq = _matmul(h, wq)
        m, lg = edge_call(ea2, src_c, dst_c, hwm, xwq, we)
        lgs = lg - jnp.max(lg)
        h = out_call(lgs, m, dst_r, h)

    l = ligand_batch.shape[0]
    lp = ((l + _RB - 1) // _RB) * _RB
    out_l = jnp.pad(h[:l], ((0, lp - l), (0, 0)))
    lb_pad = jnp.pad(ligand_batch, (0, lp - l), constant_values=_B)
    lb3 = lb_pad.reshape(1, 1, lp)
    rows = _B * _MAXLEN

    res = pl.pallas_call(
        _scatter_kernel,
        grid=(rows // _RB,),
        in_specs=[
            pl.BlockSpec((1, 1, lp), lambda i: (0, 0, 0)),
            pl.BlockSpec((lp, d), lambda i: (0, 0)),
        ],
        out_specs=pl.BlockSpec((_RB, d), lambda i: (i, 0)),
        out_shape=jax.ShapeDtypeStruct((rows, d), jnp.float32),
    )(lb3, out_l)
    return res.reshape(_B, _MAXLEN, d)


# speedup vs baseline: 1.7225x; 1.7225x over previous
import math

import jax
import jax.numpy as jnp
from jax.experimental import pallas as pl
from jax.experimental.pallas import tpu as pltpu

_G = 4
_MAXLEN = 128
_B = 64
_NB = 1000
_EBE = 512
_EBO = 512
_RB = 512


def _mm_kernel(x_ref, w_ref, o_ref):
    o_ref[...] = jnp.dot(x_ref[...], w_ref[...],
                         preferred_element_type=jnp.float32)


def _matmul(x, w):
    return pl.pallas_call(
        _mm_kernel,
        out_shape=jax.ShapeDtypeStruct((x.shape[0], w.shape[1]), jnp.float32),
    )(x, w)


def _group_mat(d, dtype=jnp.float32):
    dg = d // _G
    gi = jax.lax.broadcasted_iota(jnp.int32, (d, _G), 0) // dg
    gj = jax.lax.broadcasted_iota(jnp.int32, (d, _G), 1)
    return (gi == gj).astype(dtype)


def _edge_kernel(ea_ref, src_ref, dst_ref, hwm_ref, xwq_ref, we_ref,
                 m_ref, lg_ref):
    n = hwm_ref.shape[0]
    eb = ea_ref.shape[0]
    d = hwm_ref.shape[1]
    srcv = src_ref[0]
    dstv = dst_ref[0]
    iota_n = jax.lax.broadcasted_iota(jnp.int32, (eb, n), 1)
    oh_s = (iota_n == srcv).astype(jnp.float32)
    oh_d = (iota_n == dstv).astype(jnp.float32)
    e = jnp.dot(ea_ref[...], we_ref[...], preferred_element_type=jnp.float32)
    m = jnp.dot(oh_s, hwm_ref[...], preferred_element_type=jnp.float32) + e
    q = jnp.dot(oh_d, xwq_ref[...], preferred_element_type=jnp.float32)
    m_ref[...] = m
    rt = _group_mat(d)
    lg_ref[...] = jnp.dot(m * q, rt, preferred_element_type=jnp.float32) * (
        1.0 / math.sqrt(d // _G))


def _out_kernel(lg_ref, m_ref, dst_ref, h_ref, o_ref, sacc_ref, aacc_ref):
    e_idx = pl.program_id(1)
    n_e = pl.num_programs(1)
    nb = o_ref.shape[0]
    eb = m_ref.shape[0]
    d = m_ref.shape[1]

    @pl.when(e_idx == 0)
    def _():
        sacc_ref[...] = jnp.zeros_like(sacc_ref)
        aacc_ref[...] = jnp.zeros_like(aacc_ref)

    base = pl.program_id(0) * nb
    dstv = dst_ref[0]
    iota_nb = jax.lax.broadcasted_iota(jnp.int32, (nb, eb), 0)
    oh = (iota_nb == (dstv - base)).astype(jnp.float32)
    ex = jnp.exp(lg_ref[...])
    r = _group_mat(d)
    w = m_ref[...] * jnp.dot(ex, r.T, preferred_element_type=jnp.float32)
    aacc_ref[...] += jnp.dot(oh, w, preferred_element_type=jnp.float32)
    sacc_ref[...] += jnp.dot(oh, ex, preferred_element_type=jnp.float32)

    @pl.when(e_idx == n_e - 1)
    def _():
        sfull = jnp.dot(sacc_ref[...], r.T,
                        preferred_element_type=jnp.float32)
        out = aacc_ref[...] / (sfull + 1e-30)
        o_ref[...] = jnp.maximum(out + h_ref[...], 0.0)


def _scatter_kernel(lb_ref, v_ref, o_ref):
    lp = v_ref.shape[0]
    rb = o_ref.shape[0]
    base = pl.program_id(0) * rb
    lbv = lb_ref[0]
    bio = jax.lax.broadcasted_iota(jnp.int32, (_B, lp), 0)
    p = bio == lbv
    colidx = jax.lax.broadcasted_iota(jnp.int32, (_B, lp), 1)
    fo = jnp.min(jnp.where(p, colidx, lp), axis=1, keepdims=True)
    forow = jnp.sum(jnp.where(p, fo, 0), axis=0, keepdims=True)
    idxrow = jax.lax.broadcasted_iota(jnp.int32, (1, lp), 1)
    pos = idxrow - forow
    valid = (lbv < _B) & (pos < _MAXLEN)
    r = jnp.where(valid, lbv * _MAXLEN + pos, -1)
    rio = jax.lax.broadcasted_iota(jnp.int32, (rb, lp), 0) + base
    oh = (rio == r).astype(jnp.float32)
    o_ref[...] = jnp.dot(oh, v_ref[...], preferred_element_type=jnp.float32)


def kernel(x, edge_index, edge_attr, ligand_batch,
           Wm0, We0, Wq0, Wm1, We1, Wq1, Wm2, We2, Wq2):
    n, d = x.shape
    src, dst = edge_index[0], edge_index[1]
    src2 = jnp.concatenate([src, dst])
    dst2 = jnp.concatenate([dst, src])
    ea2 = jnp.concatenate([edge_attr, edge_attr], axis=0)
    e2 = src2.shape[0]
    de = ea2.shape[1]

    nbe = e2 // _EBE
    nbo = e2 // _EBO
    nbn = n // _NB
    src_c = src2.reshape(nbe, _EBE, 1)
    dst_c = dst2.reshape(nbe, _EBE, 1)
    dst_r = dst2.reshape(nbo, 1, _EBO)

    edge_call = pl.pallas_call(
        _edge_kernel,
        grid=(nbe,),
        in_specs=[
            pl.BlockSpec((_EBE, de), lambda i: (i, 0)),
            pl.BlockSpec((1, _EBE, 1), lambda i: (i, 0, 0)),
            pl.BlockSpec((1, _EBE, 1), lambda i: (i, 0, 0)),
            pl.BlockSpec((n, d), lambda i: (0, 0)),
            pl.BlockSpec((n, d), lambda i: (0, 0)),
            pl.BlockSpec((de, d), lambda i: (0, 0)),
        ],
        out_specs=[
            pl.BlockSpec((_EBE, d), lambda i: (i, 0)),
            pl.BlockSpec((_EBE, _G), lambda i: (i, 0)),
        ],
        out_shape=[
            jax.ShapeDtypeStruct((e2, d), jnp.float32),
            jax.ShapeDtypeStruct((e2, _G), jnp.float32),
        ],
    )

    out_call = pl.pallas_call(
        _out_kernel,
        grid=(nbn, nbo),
        in_specs=[
            pl.BlockSpec((_EBO, _G), lambda i, j: (j, 0)),
            pl.BlockSpec((_EBO, d), lambda i, j: (j, 0)),
            pl.BlockSpec((1, 1, _EBO), lambda i, j: (j, 0, 0)),
            pl.BlockSpec((_NB, d), lambda i, j: (i, 0)),
        ],
        out_specs=pl.BlockSpec((_NB, d), lambda i, j: (i, 0)),
        out_shape=jax.ShapeDtypeStruct((n, d), jnp.float32),
        scratch_shapes=[
            pltpu.VMEM((_NB, _G), jnp.float32),
            pltpu.VMEM((_NB, d), jnp.float32),
        ],
    )

    h = x
    for wm, we, wq in ((Wm0, We0, Wq0), (Wm1, We1, Wq1), (Wm2, We2, Wq2)):
        hwm = _matmul(h, wm)
        xwq = _matmul(h, wq)
        m, lg = edge_call(ea2, src_c, dst_c, hwm, xwq, we)
        lgs = lg - jnp.max(lg)
        h = out_call(lgs, m, dst_r, h)

    l = ligand_batch.shape[0]
    lp = ((l + _RB - 1) // _RB) * _RB
    out_l = jnp.pad(h[:l], ((0, lp - l), (0, 0)))
    lb_pad = jnp.pad(ligand_batch, (0, lp - l), constant_values=_B)
    lb3 = lb_pad.reshape(1, 1, lp)
    rows = _B * _MAXLEN

    res = pl.pallas_call(
        _scatter_kernel,
        grid=(rows // _RB,),
        in_specs=[
            pl.BlockSpec((1, 1, lp), lambda i: (0, 0, 0)),
            pl.BlockSpec((lp, d), lambda i: (0, 0)),
        ],
        out_specs=pl.BlockSpec((_RB, d), lambda i: (i, 0)),
        out_shape=jax.ShapeDtypeStruct((rows, d), jnp.float32),
    )(lb3, out_l)
    return res.reshape(_B, _MAXLEN, d)
